# baseline (speedup 1.0000x reference)
import functools

import jax
import jax.numpy as jnp
from jax import lax
from jax.experimental import pallas as pl
from jax.experimental.pallas import tpu as pltpu
from jax.experimental.pallas import tpu_sc as plsc

DIM = 64
SCALE = 8.0

_NC = 2
_NS = 16
_NW = _NC * _NS

_C = 128
_NB = 4
_TCB = 512


@functools.partial(jax.jit, static_argnums=(2, 3))
def _lookup(table128, x_t, l_seq, b_total):
    stripe = b_total // _NW
    per_l = stripe // _C
    assert per_l == _NB
    mesh = plsc.VectorSubcoreMesh(core_axis_name="c", subcore_axis_name="s")

    @functools.partial(
        pl.kernel,
        mesh=mesh,
        out_type=jax.ShapeDtypeStruct((l_seq, b_total, 128), jnp.float32),
        scratch_types=[
            [pltpu.VMEM((_C,), jnp.int32) for _ in range(_NB)],
            [pltpu.VMEM((_C, 128), jnp.float32) for _ in range(_NB)],
            [pltpu.SemaphoreType.DMA for _ in range(_NB)],
            [pltpu.SemaphoreType.DMA for _ in range(_NB)],
            [pltpu.SemaphoreType.DMA for _ in range(_NB)],
        ],
        compiler_params=pltpu.CompilerParams(needs_layout_passes=False),
    )
    def k(tab_hbm, xt_hbm, out_hbm, idx_v, pairs_v, isem, gsem, ssem):
        wid = lax.axis_index("s") * _NC + lax.axis_index("c")
        b0w = wid * stripe

        def start_idx(l, slot):
            pltpu.async_copy(
                xt_hbm.at[l, pl.ds(b0w + slot * _C, _C)], idx_v[slot], isem[slot]
            )

        def wait_idx(slot):
            pltpu.make_async_copy(
                xt_hbm.at[0, pl.ds(b0w + slot * _C, _C)], idx_v[slot], isem[slot]
            ).wait()

        def start_gather(slot):
            pltpu.async_copy(tab_hbm.at[idx_v[slot]], pairs_v[slot], gsem[slot])

        def wait_gather(slot):
            pltpu.make_async_copy(
                tab_hbm.at[idx_v[slot]], pairs_v[slot], gsem[slot]
            ).wait()

        def start_store(l, slot):
            pltpu.async_copy(
                pairs_v[slot],
                out_hbm.at[l, pl.ds(b0w + slot * _C, _C)],
                ssem[slot],
            )

        def wait_store(l, slot):
            pltpu.make_async_copy(
                pairs_v[slot],
                out_hbm.at[l, pl.ds(b0w + slot * _C, _C)],
                ssem[slot],
            ).wait()

        for j in range(_NB):
            start_idx(0, j)
        for j in range(_NB - 1):
            wait_idx(j)
            start_gather(j)

        def outer(l, carry):
            for b in range(_NB):
                wait_gather(b)

                @pl.when(l < l_seq - 1)
                def _():
                    start_idx(l + 1, b)

                nslot = (b + 3) % _NB
                if b == 0:
                    @pl.when(l >= 1)
                    def _():
                        wait_store(l - 1, nslot)

                    wait_idx(nslot)
                    start_gather(nslot)
                else:
                    wait_store(l, nslot)

                    @pl.when(l < l_seq - 1)
                    def _():
                        wait_idx(nslot)
                        start_gather(nslot)

                start_store(l, b)
            return carry

        lax.fori_loop(0, l_seq, outer, 0)

        wait_store(l_seq - 1, _NB - 1)

    return k(table128, x_t)


@functools.partial(jax.jit, static_argnums=(1, 2))
def _format(rows, l_seq, b_total):
    def body(in_ref, out_ref):
        blk = in_ref[0][:, :DIM]
        out_ref[0] = jnp.transpose(blk, (1, 0)) * SCALE

    return pl.pallas_call(
        body,
        grid=(l_seq, b_total // _TCB),
        in_specs=[
            pl.BlockSpec((1, _TCB, 128), lambda l, j: (l, j, 0)),
        ],
        out_specs=pl.BlockSpec((1, DIM, _TCB), lambda l, j: (l, 0, j)),
        out_shape=jax.ShapeDtypeStruct((l_seq, DIM, b_total), jnp.float32),
    )(rows)


def kernel(x, table):
    b, l = x.shape
    x_t = jnp.transpose(x).astype(jnp.int32)
    table128 = jnp.pad(table, ((0, 0), (0, 128 - DIM)))
    rows = _lookup(table128, x_t, l, b)
    res = _format(rows, l, b)
    return jnp.transpose(res, (2, 0, 1))


# speedup vs baseline: 1.7200x; 1.7200x over previous
import functools

import jax
import jax.numpy as jnp
from jax import lax
from jax.experimental import pallas as pl
from jax.experimental.pallas import tpu as pltpu
from jax.experimental.pallas import tpu_sc as plsc

DIM = 64
SCALE = 8.0

_NC = 2
_NS = 16
_NW = _NC * _NS

_C = 128
_NB = 4
_TCB = 512


@functools.partial(jax.jit, static_argnums=(2, 3))
def _lookup(table128, x_t, l_seq, b_total):
    stripe = b_total // _NW
    per_l = stripe // _C
    assert per_l == _NB
    mesh = plsc.VectorSubcoreMesh(core_axis_name="c", subcore_axis_name="s")

    @functools.partial(
        pl.kernel,
        mesh=mesh,
        out_type=jax.ShapeDtypeStruct((l_seq, b_total, 128), jnp.float32),
        scratch_types=[
            [pltpu.VMEM((_C,), jnp.int32) for _ in range(_NB)],
            [pltpu.VMEM((_C, 128), jnp.float32) for _ in range(_NB)],
            [pltpu.SemaphoreType.DMA for _ in range(_NB)],
            [pltpu.SemaphoreType.DMA for _ in range(_NB)],
            [pltpu.SemaphoreType.DMA for _ in range(_NB)],
        ],
        compiler_params=pltpu.CompilerParams(needs_layout_passes=False),
    )
    def k(tab_hbm, xt_hbm, out_hbm, idx_v, pairs_v, isem, gsem, ssem):
        wid = lax.axis_index("s") * _NC + lax.axis_index("c")
        b0w = wid * stripe

        def start_idx(l, slot):
            pltpu.async_copy(
                xt_hbm.at[l, pl.ds(b0w + slot * _C, _C)], idx_v[slot], isem[slot]
            )

        def wait_idx(slot):
            pltpu.make_async_copy(
                xt_hbm.at[0, pl.ds(b0w + slot * _C, _C)], idx_v[slot], isem[slot]
            ).wait()

        def start_gather(slot):
            pltpu.async_copy(tab_hbm.at[idx_v[slot]], pairs_v[slot], gsem[slot])

        def wait_gather(slot):
            pltpu.make_async_copy(
                tab_hbm.at[idx_v[slot]], pairs_v[slot], gsem[slot]
            ).wait()

        def start_store(l, slot):
            pltpu.async_copy(
                pairs_v[slot],
                out_hbm.at[l, pl.ds(b0w + slot * _C, _C)],
                ssem[slot],
            )

        def wait_store(l, slot):
            pltpu.make_async_copy(
                pairs_v[slot],
                out_hbm.at[l, pl.ds(b0w + slot * _C, _C)],
                ssem[slot],
            ).wait()

        for j in range(_NB):
            start_idx(0, j)
        for j in range(_NB - 1):
            wait_idx(j)
            start_gather(j)

        def outer(l, carry):
            for b in range(_NB):
                wait_gather(b)

                @pl.when(l < l_seq - 1)
                def _():
                    start_idx(l + 1, b)

                nslot = (b + 3) % _NB
                if b == 0:
                    @pl.when(l >= 1)
                    def _():
                        wait_store(l - 1, nslot)

                    wait_idx(nslot)
                    start_gather(nslot)
                else:
                    wait_store(l, nslot)

                    @pl.when(l < l_seq - 1)
                    def _():
                        wait_idx(nslot)
                        start_gather(nslot)

                start_store(l, b)
            return carry

        lax.fori_loop(0, l_seq, outer, 0)

        wait_store(l_seq - 1, _NB - 1)

    return k(table128, x_t)


@functools.partial(jax.jit, static_argnums=(1, 2))
def _format(rows, l_seq, b_total):
    def body(in_ref, out_ref):
        blk = in_ref[0][:, :DIM]
        out_ref[0] = jnp.transpose(blk, (1, 0)) * SCALE

    return pl.pallas_call(
        body,
        grid=(l_seq, b_total // _TCB),
        in_specs=[
            pl.BlockSpec((1, _TCB, 128), lambda l, j: (l, j, 0)),
        ],
        out_specs=pl.BlockSpec((1, DIM, _TCB), lambda l, j: (l, 0, j)),
        out_shape=jax.ShapeDtypeStruct((l_seq, DIM, b_total), jnp.float32),
    )(rows)


def kernel(x, table):
    b, l = x.shape
    x_t = jnp.transpose(x).astype(jnp.int32)
    table128 = jnp.pad(table, ((0, 0), (0, 128 - DIM)))
    rows = _lookup(table128, x_t, l, b)
    return jnp.transpose(rows[:, :, :DIM], (1, 0, 2)) * SCALE


# speedup vs baseline: 1.7214x; 1.0008x over previous
import functools

import jax
import jax.numpy as jnp
from jax import lax
from jax.experimental import pallas as pl
from jax.experimental.pallas import tpu as pltpu
from jax.experimental.pallas import tpu_sc as plsc

DIM = 64
SCALE = 8.0

_NC = 2
_NS = 16
_NW = _NC * _NS

_C = 128
_NB = 4


@functools.partial(jax.jit, static_argnums=(2, 3))
def _lookup(table128, x_t, l_seq, b_total):
    stripe = b_total // _NW
    per_l = stripe // _C
    assert per_l == _NB
    mesh = plsc.VectorSubcoreMesh(core_axis_name="c", subcore_axis_name="s")

    @functools.partial(
        pl.kernel,
        mesh=mesh,
        out_type=jax.ShapeDtypeStruct((l_seq, b_total, 128), jnp.float32),
        scratch_types=[
            [pltpu.VMEM((_C,), jnp.int32) for _ in range(_NB)],
            [pltpu.VMEM((_C, 128), jnp.float32) for _ in range(_NB)],
            [pltpu.SemaphoreType.DMA for _ in range(_NB)],
            [pltpu.SemaphoreType.DMA for _ in range(_NB)],
            [pltpu.SemaphoreType.DMA for _ in range(_NB)],
        ],
        compiler_params=pltpu.CompilerParams(needs_layout_passes=False),
    )
    def k(tab_hbm, xt_hbm, out_hbm, idx_v, pairs_v, isem, gsem, ssem):
        wid = lax.axis_index("s") * _NC + lax.axis_index("c")
        b0w = wid * stripe

        def start_idx(l, slot):
            pltpu.async_copy(
                xt_hbm.at[l, pl.ds(b0w + slot * _C, _C)], idx_v[slot], isem[slot]
            )

        def wait_idx(slot):
            pltpu.make_async_copy(
                xt_hbm.at[0, pl.ds(b0w + slot * _C, _C)], idx_v[slot], isem[slot]
            ).wait()

        def start_gather(slot):
            pltpu.async_copy(tab_hbm.at[idx_v[slot]], pairs_v[slot], gsem[slot])

        def wait_gather(slot):
            pltpu.make_async_copy(
                tab_hbm.at[idx_v[slot]], pairs_v[slot], gsem[slot]
            ).wait()

        def start_store(l, slot):
            pltpu.async_copy(
                pairs_v[slot],
                out_hbm.at[l, pl.ds(b0w + slot * _C, _C)],
                ssem[slot],
            )

        def wait_store(l, slot):
            pltpu.make_async_copy(
                pairs_v[slot],
                out_hbm.at[l, pl.ds(b0w + slot * _C, _C)],
                ssem[slot],
            ).wait()

        for j in range(_NB):
            start_idx(0, j)
        for j in range(_NB - 1):
            wait_idx(j)
            start_gather(j)

        def outer(l, carry):
            for b in range(_NB):
                wait_gather(b)

                @pl.when(l < l_seq - 1)
                def _():
                    start_idx(l + 1, b)

                nslot = (b + 3) % _NB
                if b == 0:
                    @pl.when(l >= 1)
                    def _():
                        wait_store(l - 1, nslot)

                    wait_idx(nslot)
                    start_gather(nslot)
                else:
                    wait_store(l, nslot)

                    @pl.when(l < l_seq - 1)
                    def _():
                        wait_idx(nslot)
                        start_gather(nslot)

                start_store(l, b)
            return carry

        lax.fori_loop(0, l_seq, outer, 0)

        wait_store(l_seq - 1, _NB - 1)

    return k(table128, x_t)


def kernel(x, table):
    b, l = x.shape
    x_t = jnp.transpose(x).astype(jnp.int32)
    table128 = jnp.pad(table, ((0, 0), (0, 128 - DIM)))
    rows = _lookup(table128, x_t, l, b)
    return jnp.transpose(rows[:, :, :DIM], (1, 0, 2)) * SCALE
